# baseline (speedup 1.0000x reference)
import jax
import jax.numpy as jnp
from jax.experimental import pallas as pl
from jax.experimental.pallas import tpu as pltpu

_V = 100000
_K = 64
_LAMB = 0.01
_B = 256
_TV = 2000
_NT = _V // _TV


def _fused_kernel(q_ref, emb_ref, acc_ref, e0, e1, e2, sems):
    i = pl.program_id(0)
    bufs = (e0, e1, e2)

    def dma(step, slot, c):
        return pltpu.make_async_copy(
            emb_ref.at[pl.ds(step * _TV, _TV), c, :],
            bufs[c].at[slot],
            sems.at[slot, c],
        )

    @pl.when(i == 0)
    def _prologue():
        for c in range(3):
            dma(0, 0, c).start()
        acc_ref[...] = jnp.zeros_like(acc_ref)

    slot = jax.lax.rem(i, 2)
    nxt = jax.lax.rem(i + 1, 2)

    @pl.when(i + 1 < _NT)
    def _prefetch():
        for c in range(3):
            dma(i + 1, nxt, c).start()

    for c in range(3):
        dma(i, slot, c).wait()

    e = jnp.concatenate([e0[slot], e1[slot], e2[slot]], axis=1)
    logits = jax.lax.dot_general(
        q_ref[...], e.astype(jnp.bfloat16),
        (((1,), (1,)), ((), ())), preferred_element_type=jnp.float32)
    acc_ref[...] += jnp.sum(jnp.exp(logits), axis=1, keepdims=True)


def kernel(triples, emb):
    sub = triples[:, 0]
    pred = triples[:, 1]
    obj = triples[:, 2]

    s = jnp.take(emb, sub, axis=0)
    p = jnp.take(emb, pred, axis=0)
    o = jnp.take(emb, obj, axis=0)

    q_o = jnp.concatenate([s[:, 2] * p[:, 2], s[:, 1] * p[:, 1], s[:, 0] * p[:, 0]], axis=-1)
    q_s = jnp.concatenate([p[:, 0] * o[:, 2], p[:, 1] * o[:, 1], p[:, 2] * o[:, 0]], axis=-1)
    q = jnp.concatenate([q_o, q_s], axis=0).astype(jnp.bfloat16)

    acc = pl.pallas_call(
        _fused_kernel,
        grid=(_NT,),
        in_specs=[
            pl.BlockSpec((2 * _B, 3 * _K), lambda i: (0, 0)),
            pl.BlockSpec(memory_space=pltpu.MemorySpace.HBM),
        ],
        out_specs=pl.BlockSpec((2 * _B, 1), lambda i: (0, 0)),
        out_shape=jax.ShapeDtypeStruct((2 * _B, 1), jnp.float32),
        scratch_shapes=[
            pltpu.VMEM((2, _TV, _K), jnp.float32),
            pltpu.VMEM((2, _TV, _K), jnp.float32),
            pltpu.VMEM((2, _TV, _K), jnp.float32),
            pltpu.SemaphoreType.DMA((2, 3)),
        ],
    )(q, emb)

    score = jnp.sum(s[:, 0] * p[:, 0] * o[:, 2]
                    + s[:, 1] * p[:, 1] * o[:, 1]
                    + s[:, 2] * p[:, 2] * o[:, 0], axis=-1)
    es = jnp.exp(score)
    lse_o = jnp.log(acc[:_B, 0] - es)
    lse_s = jnp.log(acc[_B:, 0] - es)
    reg = (_LAMB / 3.0) * jnp.sum(jnp.abs(s) ** 3 + jnp.abs(p) ** 3 + jnp.abs(o) ** 3,
                                  axis=(1, 2))
    total_loss = jnp.sum(-2.0 * score + lse_o + lse_s + reg)
    return score, total_loss
